# baseline (speedup 1.0000x reference)
import jax
import jax.numpy as jnp
from jax import lax
from jax.experimental import pallas as pl
from jax.experimental.pallas import tpu as pltpu
from jax.experimental.pallas import tpu_sc as plsc

N = 10000
DIN = 128
H = 128
E = 640000
W = 64
WC = 16
N_PAD = 10240
NS = 16
CHUNK = 128
KB = 4
NBLK = 79
NCHUNK = KB * NBLK
E_PAD = NS * NCHUNK * CHUNK
RB = 6
ROWS_PER_TILE = N_PAD // NS
BLK = 1000


def _proj_body(x_ref, w_ref, oa_ref, ob_ref):
    x = x_ref[0]
    w = w_ref[0]
    xw = lax.dot_general(x, w, (((1,), (1,)), ((), ())),
                         preferred_element_type=jnp.float32)
    oa_ref[0] = xw[:, :W]
    ob_ref[0] = xw[:, W:]


def _sc_body(ta_hbm, tb_hbm, idx_hbm, zros_hbm, zrosc_hbm, ones_hbm,
             out_hbm, outc_hbm,
             sidx_v, didx_v, rows_v, ones_v, acc_sh, cnt_sh,
             gsem, ssem, isem, csem):
    cid = lax.axis_index("c")
    tid = lax.axis_index("s")
    base = tid * ROWS_PER_TILE
    ws = cid * NS + tid
    wd = 2 * NS + ws

    def wait_g():
        pltpu.make_async_copy(zros_hbm, rows_v.at[0], gsem).wait()

    def wait_s():
        pltpu.make_async_copy(zros_hbm, rows_v.at[0], ssem).wait()

    def wait_i():
        pltpu.make_async_copy(idx_hbm.at[0, 0], sidx_v.at[0], isem).wait()
        pltpu.make_async_copy(idx_hbm.at[0, 0], sidx_v.at[0], isem).wait()

    def stage(b, slot):
        pltpu.async_copy(idx_hbm.at[ws, b], sidx_v.at[slot], isem)
        pltpu.async_copy(idx_hbm.at[wd, b], didx_v.at[slot], isem)

    def zero_own_rows():
        pltpu.sync_copy(zros_hbm, rows_v.at[0])
        for k in range(ROWS_PER_TILE // CHUNK):
            pltpu.sync_copy(rows_v.at[0],
                            acc_sh.at[pl.ds(base + k * CHUNK, CHUNK)])

    zero_own_rows()
    pltpu.sync_copy(zrosc_hbm, ones_v)
    for k in range(ROWS_PER_TILE // CHUNK):
        pltpu.sync_copy(ones_v, cnt_sh.at[pl.ds(base + k * CHUNK, CHUNK)])
    pltpu.sync_copy(ones_hbm, ones_v)
    plsc.subcore_barrier()

    for p, table in enumerate((ta_hbm, tb_hbm)):
        pltpu.sync_copy(idx_hbm.at[ws, 0], sidx_v.at[0])
        pltpu.sync_copy(idx_hbm.at[wd, 0], didx_v.at[0])
        for k in range(KB):
            pltpu.async_copy(table.at[sidx_v.at[0, k]], rows_v.at[k], gsem)
        stage(1, 1)
        stage(2, 2)
        for k in range(KB):
            wait_g()
            if k == 0:
                wait_i()
            if k >= 2:
                wait_s()
            pltpu.async_copy(table.at[sidx_v.at[1, k]],
                             rows_v.at[(k + KB) % RB], gsem)
            pltpu.async_copy(rows_v.at[k], acc_sh.at[didx_v.at[0, k]],
                             ssem, add=True)
            if p == 0:
                pltpu.async_copy(ones_v, cnt_sh.at[didx_v.at[0, k]],
                                 csem, add=True)

        def block(b, carry):
            slot = lax.rem(b, 4)
            nslot = lax.rem(b + 1, 4)
            not_last = b < NBLK - 1

            @pl.when(b < NBLK - 2)
            def _():
                stage(b + 2, lax.rem(b + 2, 4))

            for k in range(KB):
                i = b * KB + k
                buf = lax.rem(i, RB)
                nbuf = lax.rem(i + KB, RB)
                wait_g()
                if k == 0:
                    @pl.when(not_last)
                    def _():
                        wait_i()
                wait_s()

                @pl.when(not_last)
                def _():
                    pltpu.async_copy(table.at[sidx_v.at[nslot, k]],
                                     rows_v.at[nbuf], gsem)

                pltpu.async_copy(rows_v.at[buf],
                                 acc_sh.at[didx_v.at[slot, k]],
                                 ssem, add=True)
                if p == 0:
                    pltpu.async_copy(ones_v,
                                     cnt_sh.at[didx_v.at[slot, k]],
                                     csem, add=True)
            return carry

        lax.fori_loop(1, NBLK, block, 0, unroll=False)
        wait_s()
        wait_s()
        if p == 0:
            def drain(_, carry):
                pltpu.make_async_copy(ones_hbm, ones_v, csem).wait()
                return carry
            lax.fori_loop(0, NCHUNK, drain, 0, unroll=False)
        plsc.subcore_barrier()
        for k in range(ROWS_PER_TILE // CHUNK):
            sl = pl.ds(base + k * CHUNK, CHUNK)
            pltpu.sync_copy(acc_sh.at[sl], rows_v.at[0])
            pltpu.sync_copy(rows_v.at[0], out_hbm.at[cid, p].at[sl])
        if p == 0:
            for k in range(ROWS_PER_TILE // CHUNK):
                sl = pl.ds(base + k * CHUNK, CHUNK)
                pltpu.sync_copy(cnt_sh.at[sl], ones_v)
                pltpu.sync_copy(ones_v, outc_hbm.at[cid].at[sl])
            zero_own_rows()
            pltpu.sync_copy(ones_hbm, ones_v)
            plsc.subcore_barrier()


def _tgt_body(x_ref, w_ref, o_ref):
    o_ref[0] = lax.dot_general(x_ref[0], w_ref[0], (((1,), (1,)), ((), ())),
                               preferred_element_type=jnp.float32)


def _gru_body(tgt_ref, acca_ref, accb_ref, cnt_ref, wih_ref, whh_ref,
              bih_ref, bhh_ref, o_ref):
    tgt = tgt_ref[0]
    aggsum = jnp.concatenate([acca_ref[0], accb_ref[0]], axis=1)
    agg = aggsum / jnp.maximum(cnt_ref[0][:, :1], 1.0)
    gi = lax.dot_general(tgt, wih_ref[0], (((1,), (1,)), ((), ())),
                         preferred_element_type=jnp.float32) + bih_ref[0, 0]
    gh = lax.dot_general(agg, whh_ref[0], (((1,), (1,)), ((), ())),
                         preferred_element_type=jnp.float32) + bhh_ref[0, 0]
    r = jax.nn.sigmoid(gi[:, :H] + gh[:, :H])
    z = jax.nn.sigmoid(gi[:, H:2 * H] + gh[:, H:2 * H])
    n = jnp.tanh(gi[:, 2 * H:] + r * gh[:, 2 * H:])
    o_ref[0] = jax.nn.relu((1.0 - z) * n + z * agg)


def kernel(x_spot, x_user, ei_user_spot, ei_spot_user,
           W_src_us, W_tgt_us, W_src_su, W_tgt_su,
           Wih_us, Whh_us, bih_us, bhh_us,
           Wih_su, Whh_su, bih_su, bhh_su):
    f32 = jnp.float32
    nb = N // BLK

    x_src = jnp.stack([x_user, x_spot])
    w_src = jnp.stack([W_src_us, W_src_su])
    table_a, table_b = pl.pallas_call(
        _proj_body,
        grid=(2, nb),
        in_specs=[
            pl.BlockSpec((1, BLK, DIN), lambda b, i: (b, i, 0)),
            pl.BlockSpec((1, H, DIN), lambda b, i: (b, 0, 0)),
        ],
        out_specs=[
            pl.BlockSpec((1, BLK, W), lambda b, i: (b, i, 0)),
            pl.BlockSpec((1, BLK, W), lambda b, i: (b, i, 0)),
        ],
        out_shape=[
            jax.ShapeDtypeStruct((2, N, W), f32),
            jax.ShapeDtypeStruct((2, N, W), f32),
        ],
    )(x_src, w_src)
    table_a = table_a.reshape(2 * N, W)
    table_b = table_b.reshape(2 * N, W)

    pad = E_PAD - E
    ar = jnp.arange(pad, dtype=jnp.int32)
    pad_src = ar % (2 * N)
    pad_dst = N + ar % (N_PAD - N)
    sidx = jnp.stack([
        jnp.concatenate([ei_user_spot[0], pad_src]),
        jnp.concatenate([ei_spot_user[0] + N, pad_src]),
    ])
    didx = jnp.stack([
        jnp.concatenate([ei_user_spot[1], pad_dst]),
        jnp.concatenate([ei_spot_user[1], pad_dst]),
    ])
    idx = jnp.stack([sidx, didx]).reshape(4 * NS, NBLK, KB, CHUNK)
    zros = jnp.zeros((CHUNK, W), f32)
    zrosc = jnp.zeros((CHUNK, WC), f32)
    ones = jnp.ones((CHUNK, WC), f32)

    mesh = plsc.VectorSubcoreMesh(core_axis_name="c", subcore_axis_name="s")
    acc, cnt = pl.kernel(
        _sc_body,
        out_type=(
            jax.ShapeDtypeStruct((2, 2, N_PAD, W), f32),
            jax.ShapeDtypeStruct((2, N_PAD, WC), f32),
        ),
        mesh=mesh,
        scratch_types=[
            pltpu.VMEM((4, KB, CHUNK), jnp.int32),
            pltpu.VMEM((4, KB, CHUNK), jnp.int32),
            pltpu.VMEM((RB, CHUNK, W), f32),
            pltpu.VMEM((CHUNK, WC), f32),
            pltpu.VMEM_SHARED((N_PAD, W), f32),
            pltpu.VMEM_SHARED((N_PAD, WC), f32),
            pltpu.SemaphoreType.DMA,
            pltpu.SemaphoreType.DMA,
            pltpu.SemaphoreType.DMA,
            pltpu.SemaphoreType.DMA,
        ],
        compiler_params=pltpu.CompilerParams(use_tc_tiling_on_sc=False),
    )(table_a, table_b, idx, zros, zrosc, ones)

    x_tgt = jnp.stack([x_spot, x_user])
    w_tgt = jnp.stack([W_tgt_us, W_tgt_su])
    tgt = pl.pallas_call(
        _tgt_body,
        grid=(2, nb),
        in_specs=[
            pl.BlockSpec((1, BLK, DIN), lambda b, i: (b, i, 0)),
            pl.BlockSpec((1, H, DIN), lambda b, i: (b, 0, 0)),
        ],
        out_specs=pl.BlockSpec((1, BLK, H), lambda b, i: (b, i, 0)),
        out_shape=jax.ShapeDtypeStruct((2, N, H), f32),
    )(x_tgt, w_tgt)

    wih = jnp.stack([Wih_us, Wih_su])
    whh = jnp.stack([Whh_us, Whh_su])
    bih = jnp.stack([bih_us, bih_su]).reshape(2, 1, 3 * H)
    bhh = jnp.stack([bhh_us, bhh_su]).reshape(2, 1, 3 * H)
    out = pl.pallas_call(
        _gru_body,
        grid=(2, nb),
        in_specs=[
            pl.BlockSpec((1, BLK, H), lambda b, i: (b, i, 0)),
            pl.BlockSpec((1, BLK, W), lambda b, i: (b, i, 0)),
            pl.BlockSpec((1, BLK, W), lambda b, i: (b, i, 0)),
            pl.BlockSpec((1, BLK, WC), lambda b, i: (b, i, 0)),
            pl.BlockSpec((1, 3 * H, H), lambda b, i: (b, 0, 0)),
            pl.BlockSpec((1, 3 * H, H), lambda b, i: (b, 0, 0)),
            pl.BlockSpec((1, 1, 3 * H), lambda b, i: (b, 0, 0)),
            pl.BlockSpec((1, 1, 3 * H), lambda b, i: (b, 0, 0)),
        ],
        out_specs=pl.BlockSpec((1, BLK, H), lambda b, i: (b, i, 0)),
        out_shape=jax.ShapeDtypeStruct((2, N, H), f32),
    )(tgt, acc[:, 0, :N], acc[:, 1, :N], cnt[:, :N],
      wih, whh, bih, bhh)

    return (out[0], out[1])


# speedup vs baseline: 1.0023x; 1.0023x over previous
import jax
import jax.numpy as jnp
from jax import lax
from jax.experimental import pallas as pl
from jax.experimental.pallas import tpu as pltpu
from jax.experimental.pallas import tpu_sc as plsc

N = 10000
DIN = 128
H = 128
E = 640000
W = 64
WC = 16
N_PAD = 10240
NS = 16
CHUNK = 128
KB = 4
NBLK = 79
NCHUNK = KB * NBLK
E_PAD = NS * NCHUNK * CHUNK
RB = 6
ROWS_PER_TILE = N_PAD // NS
BLK = 1000


def _proj_body(x_ref, w_ref, oa_ref, ob_ref):
    x = x_ref[0]
    w = w_ref[0]
    xw = lax.dot_general(x, w, (((1,), (1,)), ((), ())),
                         preferred_element_type=jnp.float32)
    oa_ref[0] = xw[:, :W]
    ob_ref[0] = xw[:, W:]


def _sc_body(ta_hbm, tb_hbm, idx_hbm, zros_hbm, zrosc_hbm, ones_hbm,
             out_hbm, outc_hbm,
             sidx_v, didx_v, rows_v, ones_v, acc_sh, cnt_sh,
             gsem, ssem, isem, csem):
    cid = lax.axis_index("c")
    tid = lax.axis_index("s")
    base = tid * ROWS_PER_TILE
    ws = cid * NS + tid
    wd = 2 * NS + ws

    def wait_g():
        pltpu.make_async_copy(zros_hbm, rows_v.at[0], gsem).wait()

    def wait_s():
        pltpu.make_async_copy(zros_hbm, rows_v.at[0], ssem).wait()

    def wait_i():
        pltpu.make_async_copy(idx_hbm.at[0, 0], sidx_v.at[0], isem).wait()
        pltpu.make_async_copy(idx_hbm.at[0, 0], sidx_v.at[0], isem).wait()

    def stage(b, slot):
        pltpu.async_copy(idx_hbm.at[ws, b], sidx_v.at[slot], isem)
        pltpu.async_copy(idx_hbm.at[wd, b], didx_v.at[slot], isem)

    def zero_own_rows():
        pltpu.sync_copy(zros_hbm, rows_v.at[0])
        for k in range(ROWS_PER_TILE // CHUNK):
            pltpu.sync_copy(rows_v.at[0],
                            acc_sh.at[pl.ds(base + k * CHUNK, CHUNK)])

    zero_own_rows()
    pltpu.sync_copy(zrosc_hbm, ones_v)
    for k in range(ROWS_PER_TILE // CHUNK):
        pltpu.sync_copy(ones_v, cnt_sh.at[pl.ds(base + k * CHUNK, CHUNK)])
    pltpu.sync_copy(ones_hbm, ones_v)
    plsc.subcore_barrier()

    for p, table in enumerate((ta_hbm, tb_hbm)):
        pltpu.sync_copy(idx_hbm.at[ws, 0], sidx_v.at[0])
        pltpu.sync_copy(idx_hbm.at[wd, 0], didx_v.at[0])
        for k in range(KB):
            pltpu.async_copy(table.at[sidx_v.at[0, k]], rows_v.at[k], gsem)
        stage(1, 1)
        stage(2, 2)
        for k in range(KB):
            wait_g()
            if k == 0:
                wait_i()
            if k >= 2:
                wait_s()
            pltpu.async_copy(table.at[sidx_v.at[1, k]],
                             rows_v.at[(k + KB) % RB], gsem)
            pltpu.async_copy(rows_v.at[k], acc_sh.at[didx_v.at[0, k]],
                             ssem, add=True)
            if p == 0:
                pltpu.async_copy(ones_v, cnt_sh.at[didx_v.at[0, k]],
                                 csem, add=True)

        def block(b, carry):
            slot = lax.rem(b, 4)
            nslot = lax.rem(b + 1, 4)
            not_last = b < NBLK - 1

            @pl.when(b < NBLK - 2)
            def _():
                stage(b + 2, lax.rem(b + 2, 4))

            for k in range(KB):
                i = b * KB + k
                buf = lax.rem(i, RB)
                nbuf = lax.rem(i + KB, RB)
                wait_g()
                if k == 0:
                    @pl.when(not_last)
                    def _():
                        wait_i()
                wait_s()

                @pl.when(not_last)
                def _():
                    pltpu.async_copy(table.at[sidx_v.at[nslot, k]],
                                     rows_v.at[nbuf], gsem)

                pltpu.async_copy(rows_v.at[buf],
                                 acc_sh.at[didx_v.at[slot, k]],
                                 ssem, add=True)
                if p == 0:
                    pltpu.async_copy(ones_v,
                                     cnt_sh.at[didx_v.at[slot, k]],
                                     csem, add=True)
            return carry

        lax.fori_loop(1, NBLK, block, 0, unroll=False)
        wait_s()
        wait_s()
        if p == 0:
            def drain(_, carry):
                pltpu.make_async_copy(ones_hbm, ones_v, csem).wait()
                return carry
            lax.fori_loop(0, NCHUNK, drain, 0, unroll=False)
        plsc.subcore_barrier()
        for k in range(ROWS_PER_TILE // CHUNK):
            sl = pl.ds(base + k * CHUNK, CHUNK)
            pltpu.sync_copy(acc_sh.at[sl], rows_v.at[0])
            pltpu.sync_copy(rows_v.at[0], out_hbm.at[cid, p].at[sl])
        if p == 0:
            for k in range(ROWS_PER_TILE // CHUNK):
                sl = pl.ds(base + k * CHUNK, CHUNK)
                pltpu.sync_copy(cnt_sh.at[sl], ones_v)
                pltpu.sync_copy(ones_v, outc_hbm.at[cid].at[sl])
            zero_own_rows()
            pltpu.sync_copy(ones_hbm, ones_v)
            plsc.subcore_barrier()


def _gru_body(x_ref, wt_ref, acca_ref, accb_ref, cnt_ref, wih_ref, whh_ref,
              bih_ref, bhh_ref, o_ref):
    x = x_ref[0]
    tgt = lax.dot_general(x, wt_ref[0], (((1,), (1,)), ((), ())),
                          preferred_element_type=jnp.float32)
    aggsum = jnp.concatenate([acca_ref[0], accb_ref[0]], axis=1)
    agg = aggsum / jnp.maximum(cnt_ref[0][:, :1], 1.0)
    gi = lax.dot_general(tgt, wih_ref[0], (((1,), (1,)), ((), ())),
                         preferred_element_type=jnp.float32) + bih_ref[0, 0]
    gh = lax.dot_general(agg, whh_ref[0], (((1,), (1,)), ((), ())),
                         preferred_element_type=jnp.float32) + bhh_ref[0, 0]
    r = jax.nn.sigmoid(gi[:, :H] + gh[:, :H])
    z = jax.nn.sigmoid(gi[:, H:2 * H] + gh[:, H:2 * H])
    n = jnp.tanh(gi[:, 2 * H:] + r * gh[:, 2 * H:])
    o_ref[0] = jax.nn.relu((1.0 - z) * n + z * agg)


def kernel(x_spot, x_user, ei_user_spot, ei_spot_user,
           W_src_us, W_tgt_us, W_src_su, W_tgt_su,
           Wih_us, Whh_us, bih_us, bhh_us,
           Wih_su, Whh_su, bih_su, bhh_su):
    f32 = jnp.float32
    nb = N // BLK

    x_src = jnp.stack([x_user, x_spot])
    w_src = jnp.stack([W_src_us, W_src_su])
    table_a, table_b = pl.pallas_call(
        _proj_body,
        grid=(2, nb),
        in_specs=[
            pl.BlockSpec((1, BLK, DIN), lambda b, i: (b, i, 0)),
            pl.BlockSpec((1, H, DIN), lambda b, i: (b, 0, 0)),
        ],
        out_specs=[
            pl.BlockSpec((1, BLK, W), lambda b, i: (b, i, 0)),
            pl.BlockSpec((1, BLK, W), lambda b, i: (b, i, 0)),
        ],
        out_shape=[
            jax.ShapeDtypeStruct((2, N, W), f32),
            jax.ShapeDtypeStruct((2, N, W), f32),
        ],
    )(x_src, w_src)
    table_a = table_a.reshape(2 * N, W)
    table_b = table_b.reshape(2 * N, W)

    pad = E_PAD - E
    ar = jnp.arange(pad, dtype=jnp.int32)
    pad_src = ar % (2 * N)
    pad_dst = N + ar % (N_PAD - N)
    sidx = jnp.stack([
        jnp.concatenate([ei_user_spot[0], pad_src]),
        jnp.concatenate([ei_spot_user[0] + N, pad_src]),
    ])
    didx = jnp.stack([
        jnp.concatenate([ei_user_spot[1], pad_dst]),
        jnp.concatenate([ei_spot_user[1], pad_dst]),
    ])
    idx = jnp.stack([sidx, didx]).reshape(4 * NS, NBLK, KB, CHUNK)
    zros = jnp.zeros((CHUNK, W), f32)
    zrosc = jnp.zeros((CHUNK, WC), f32)
    ones = jnp.ones((CHUNK, WC), f32)

    mesh = plsc.VectorSubcoreMesh(core_axis_name="c", subcore_axis_name="s")
    acc, cnt = pl.kernel(
        _sc_body,
        out_type=(
            jax.ShapeDtypeStruct((2, 2, N_PAD, W), f32),
            jax.ShapeDtypeStruct((2, N_PAD, WC), f32),
        ),
        mesh=mesh,
        scratch_types=[
            pltpu.VMEM((4, KB, CHUNK), jnp.int32),
            pltpu.VMEM((4, KB, CHUNK), jnp.int32),
            pltpu.VMEM((RB, CHUNK, W), f32),
            pltpu.VMEM((CHUNK, WC), f32),
            pltpu.VMEM_SHARED((N_PAD, W), f32),
            pltpu.VMEM_SHARED((N_PAD, WC), f32),
            pltpu.SemaphoreType.DMA,
            pltpu.SemaphoreType.DMA,
            pltpu.SemaphoreType.DMA,
            pltpu.SemaphoreType.DMA,
        ],
        compiler_params=pltpu.CompilerParams(use_tc_tiling_on_sc=False),
    )(table_a, table_b, idx, zros, zrosc, ones)

    x_tgt = jnp.stack([x_spot, x_user])
    w_tgt = jnp.stack([W_tgt_us, W_tgt_su])
    wih = jnp.stack([Wih_us, Wih_su])
    whh = jnp.stack([Whh_us, Whh_su])
    bih = jnp.stack([bih_us, bih_su]).reshape(2, 1, 3 * H)
    bhh = jnp.stack([bhh_us, bhh_su]).reshape(2, 1, 3 * H)
    out = pl.pallas_call(
        _gru_body,
        grid=(2, nb),
        in_specs=[
            pl.BlockSpec((1, BLK, DIN), lambda b, i: (b, i, 0)),
            pl.BlockSpec((1, H, DIN), lambda b, i: (b, 0, 0)),
            pl.BlockSpec((1, BLK, W), lambda b, i: (b, i, 0)),
            pl.BlockSpec((1, BLK, W), lambda b, i: (b, i, 0)),
            pl.BlockSpec((1, BLK, WC), lambda b, i: (b, i, 0)),
            pl.BlockSpec((1, 3 * H, H), lambda b, i: (b, 0, 0)),
            pl.BlockSpec((1, 3 * H, H), lambda b, i: (b, 0, 0)),
            pl.BlockSpec((1, 1, 3 * H), lambda b, i: (b, 0, 0)),
            pl.BlockSpec((1, 1, 3 * H), lambda b, i: (b, 0, 0)),
        ],
        out_specs=pl.BlockSpec((1, BLK, H), lambda b, i: (b, i, 0)),
        out_shape=jax.ShapeDtypeStruct((2, N, H), f32),
    )(x_tgt, w_tgt, acc[:, 0, :N], acc[:, 1, :N], cnt[:, :N],
      wih, whh, bih, bhh)

    return (out[0], out[1])
